# Initial kernel scaffold; baseline (speedup 1.0000x reference)
import jax
import jax.numpy as jnp
from jax.experimental import pallas as pl


def kernel(img_lr, h_weight, d_weight, t_weight, b_weight):
    raise NotImplementedError("write your pallas kernel here")


# speedup vs baseline: 15.9134x; 15.9134x over previous
import functools
import jax
import jax.numpy as jnp
import numpy as np
from jax import lax
from jax.experimental import pallas as pl
from jax.experimental.pallas import tpu as pltpu
from jax.experimental.pallas import tpu_sc as plsc

L = 17
N = 512
HN = 256
PW = 520
PH = 518
ROWS_PER_W = 16
SLAB_ROWS = ROWS_PER_W + 6

_OFFSETS = {
    'h': [(0, 0), (0, 1), (0, 2), (0, 3)],
    'd': [(0, 0), (1, 1), (2, 2), (3, 3)],
    't': [(0, 0), (2, 1), (3, 1), (3, 2)],
    'b': [(0, 0), (1, 2), (1, 3), (2, 3)],
}
_KTYPES = ['h', 'd', 't', 'b']


def _rot_off(dy, dx, r):
    if r == 0:
        return (dy, dx)
    if r == 1:
        return (dx, -dy)
    if r == 2:
        return (-dy, -dx)
    return (-dx, dy)


def _perm(u, v, r):
    if r == 0:
        return 2 * u + v
    if r == 1:
        return 2 * (1 - v) + u
    if r == 2:
        return 3 - 2 * u - v
    return 2 * v + 1 - u


_PASSES = []
for _ki in range(4):
    for _r in range(4):
        offs = [_rot_off(dy, dx, _r) for (dy, dx) in _OFFSETS[_KTYPES[_ki]]]
        cperm = [_perm(q // 2, q % 2, _r) for q in range(4)]
        _PASSES.append((_ki, offs, cperm))


def _sc_body(p_hbm, *rest):
    tabs = rest[:16]
    out_hbm = rest[16]
    slab, idxb = rest[17], rest[18]
    gaths = rest[19:35]
    accp, sem = rest[35], rest[36]

    nc = 2
    wid = lax.axis_index("s") * nc + lax.axis_index("c")
    row0 = wid * ROWS_PER_W

    for c in range(3):
        pltpu.sync_copy(
            p_hbm.at[pl.ds(c * PH * PW + row0 * PW, SLAB_ROWS * PW)], slab)

        def half_body(i, _):
            yl = i // 2
            x0 = (i % 2) * HN

            for p, (_ki, offs, _cperm) in enumerate(_PASSES):
                (ady, adx), (bdy, bdx), (cdy, cdx), (ddy, ddx) = offs

                def idx_grp(t, _, yl=yl, x0=x0, p=p, ady=ady, adx=adx,
                            bdy=bdy, bdx=bdx, cdy=cdy, cdx=cdx, ddy=ddy,
                            ddx=ddx):
                    col = x0 + t * 16
                    a = slab[pl.ds((yl + 3 + ady) * PW + col + 3 + adx, 16)]
                    b = slab[pl.ds((yl + 3 + bdy) * PW + col + 3 + bdx, 16)]
                    cc = slab[pl.ds((yl + 3 + cdy) * PW + col + 3 + cdx, 16)]
                    d = slab[pl.ds((yl + 3 + ddy) * PW + col + 3 + ddx, 16)]
                    idx = a * (L ** 3) + b * (L ** 2) + cc * L + d
                    idxb[p, pl.ds(t * 16, 16)] = idx
                    return 0

                lax.fori_loop(0, 16, idx_grp, 0, unroll=4)

            copies = []
            for p in range(16):
                for j in range(2):
                    cp = pltpu.async_copy(
                        tabs[p].at[idxb.at[p].at[pl.ds(j * 128, 128)]],
                        gaths[p].at[pl.ds(j * 128, 128)],
                        sem,
                    )
                    copies.append(cp)
            for cp in copies:
                cp.wait()

            def acc_grp(t, _):
                s = gaths[0][t, :]
                for p in range(1, 16):
                    s = s + gaths[p][t, :]
                accp[t, :] = s
                return 0

            lax.fori_loop(0, HN, acc_grp, 0, unroll=2)
            pltpu.sync_copy(accp, out_hbm.at[(c * N + row0 + yl) * 2 + i % 2])
            return 0

        lax.fori_loop(0, 2 * ROWS_PER_W, half_body, 0)


@jax.jit
def _run(p_img, *tabs):
    mesh = plsc.VectorSubcoreMesh(core_axis_name="c", subcore_axis_name="s")
    kern = functools.partial(
        pl.kernel,
        mesh=mesh,
        compiler_params=pltpu.CompilerParams(use_tc_tiling_on_sc=False),
        out_type=jax.ShapeDtypeStruct((3 * N * 2, HN, 16), jnp.float32),
        scratch_types=(
            [pltpu.VMEM((SLAB_ROWS * PW,), jnp.int32),
             pltpu.VMEM((16, HN), jnp.int32)]
            + [pltpu.VMEM((HN, 16), jnp.float32) for _ in range(16)]
            + [pltpu.VMEM((HN, 16), jnp.float32),
               pltpu.SemaphoreType.DMA]
        ),
    )(_sc_body)
    return kern(p_img, *tabs)


def kernel(img_lr, h_weight, d_weight, t_weight, b_weight):
    img = img_lr.astype(jnp.int32)[0]
    p = jnp.pad(img, ((0, 0), (3, 3), (3, 3)), mode='reflect')
    p = jnp.pad(p, ((0, 0), (0, 0), (0, PW - PH)))
    weights = [h_weight, d_weight, t_weight, b_weight]
    tabs = [
        jnp.pad(weights[ki][:, np.array(cperm)] * 0.25, ((0, 0), (0, 12)))
        for ki, _offs, cperm in _PASSES
    ]
    out = _run(p.reshape(-1), *tabs)
    out = out[..., :4].reshape(3, N, 2, HN, 2, 2)
    out = out.transpose(0, 1, 4, 2, 3, 5)
    return out.reshape(1, 3, 2 * N, 2 * N)
